# Initial kernel scaffold; baseline (speedup 1.0000x reference)
import jax
import jax.numpy as jnp
from jax.experimental import pallas as pl


def kernel(dynamic_indices, dynamic_values, dynamic_values_mask, embed_table):
    raise NotImplementedError("write your pallas kernel here")


# speedup vs baseline: 1.9080x; 1.9080x over previous
import functools

import jax
import jax.numpy as jnp
from jax import lax
from jax.experimental import pallas as pl
from jax.experimental.pallas import tpu as pltpu
from jax.experimental.pallas import tpu_sc as plsc

B, L, F, D = 1024, 50, 26, 64
N = B * L
NC, NS, LANES = 2, 16, 16
NW = NC * NS
BAGS_PER_W = N // NW
NB = 16
CH = NB * F
NCHUNK = BAGS_PER_W // NB
GS = 104
NG = CH // GS


def _body(idx_hbm, val_hbm, msk_hbm, table_hbm, out_hbm,
          idx_v, val_v, msk_v, w_v, rows_v, out_v, gsem):
    wid = lax.axis_index("s") * NC + lax.axis_index("c")
    w_base = wid * BAGS_PER_W

    def chunk_body(c, _):
        bag0 = w_base + c * NB
        r0 = bag0 * F
        pltpu.sync_copy(idx_hbm.at[pl.ds(r0, CH)], idx_v)
        copies = [
            pltpu.async_copy(table_hbm.at[idx_v.at[pl.ds(g * GS, GS)]],
                             rows_v.at[pl.ds(g * GS, GS)], gsem)
            for g in range(NG)
        ]
        pltpu.sync_copy(val_hbm.at[pl.ds(r0, CH)], val_v)
        pltpu.sync_copy(msk_hbm.at[pl.ds(r0, CH)], msk_v)

        def wbody(i, _):
            s = pl.ds(i * LANES, LANES)
            m = msk_v[s]
            w = m * val_v[s] + (1.0 - m)
            w_v[s] = jnp.where(idx_v[s] != 0, w, 0.0)
            return 0

        lax.fori_loop(0, CH // LANES, wbody, 0)
        for cp in copies:
            cp.wait()

        def bag_body(b, _):
            row0 = b * F
            wv0 = w_v[pl.ds(row0, LANES)]
            wv1 = w_v[pl.ds(row0 + LANES, LANES)]
            acc = [jnp.zeros((LANES,), jnp.float32) for _ in range(4)]
            for f in range(F):
                ws = wv0[f] if f < LANES else wv1[f - LANES]
                for d in range(4):
                    acc[d] = acc[d] + ws * rows_v[row0 + f, pl.ds(d * LANES, LANES)]
            for d in range(4):
                out_v[b, pl.ds(d * LANES, LANES)] = acc[d]
            return 0

        lax.fori_loop(0, NB, bag_body, 0)
        pltpu.sync_copy(out_v, out_hbm.at[pl.ds(bag0, NB)])
        return 0

    lax.fori_loop(0, NCHUNK, chunk_body, 0)


@jax.jit
def _embed_bag(idx1, val1, msk1, table):
    mesh = plsc.VectorSubcoreMesh(core_axis_name="c", subcore_axis_name="s")
    run = functools.partial(
        pl.kernel,
        mesh=mesh,
        compiler_params=pltpu.CompilerParams(use_tc_tiling_on_sc=False),
        out_type=jax.ShapeDtypeStruct((N, D), jnp.float32),
        scratch_types=[
            pltpu.VMEM((CH,), jnp.int32),
            pltpu.VMEM((CH,), jnp.float32),
            pltpu.VMEM((CH,), jnp.float32),
            pltpu.VMEM((CH + LANES,), jnp.float32),
            pltpu.VMEM((CH, D), jnp.float32),
            pltpu.VMEM((NB, D), jnp.float32),
            pltpu.SemaphoreType.DMA,
        ],
    )(_body)
    return run(idx1, val1, msk1, table)


def kernel(dynamic_indices, dynamic_values, dynamic_values_mask, embed_table):
    idx1 = dynamic_indices.reshape(-1).astype(jnp.int32)
    val1 = dynamic_values.reshape(-1)
    msk1 = dynamic_values_mask.reshape(-1).astype(jnp.float32)
    out = _embed_bag(idx1, val1, msk1, embed_table)
    return out.reshape(B, L, D)


# speedup vs baseline: 1.9762x; 1.0357x over previous
import functools

import jax
import jax.numpy as jnp
from jax import lax
from jax.experimental import pallas as pl
from jax.experimental.pallas import tpu as pltpu
from jax.experimental.pallas import tpu_sc as plsc

B, L, F, D = 1024, 50, 26, 64
N = B * L
NC, NS, LANES = 2, 16, 16
NW = NC * NS
BAGS_PER_W = N // NW
NB = 16
CH = NB * F
NCHUNK = BAGS_PER_W // NB
GS = 104
NG = CH // GS


def _body(idx_hbm, val_hbm, msk_hbm, table_hbm, out_hbm,
          idx_v, val_v, msk_v, w_v, rows_v, out_v, sem0, sem1):
    wid = lax.axis_index("s") * NC + lax.axis_index("c")
    w_base = wid * BAGS_PER_W
    sems = (sem0, sem1)

    def load_and_fire(c, buf):
        r0 = (w_base + c * NB) * F
        pltpu.sync_copy(idx_hbm.at[pl.ds(r0, CH)], idx_v.at[buf])
        for g in range(NG):
            pltpu.async_copy(table_hbm.at[idx_v.at[buf, pl.ds(g * GS, GS)]],
                             rows_v.at[buf, pl.ds(g * GS, GS)], sems[buf])
        pltpu.sync_copy(val_hbm.at[pl.ds(r0, CH)], val_v)
        pltpu.sync_copy(msk_hbm.at[pl.ds(r0, CH)], msk_v)

        def wbody(i, _):
            s = pl.ds(i * LANES, LANES)
            m = msk_v[s]
            w = m * val_v[s] + (1.0 - m)
            w_v[buf, s] = jnp.where(idx_v[buf, s] != 0, w, 0.0)
            return 0

        lax.fori_loop(0, CH // LANES, wbody, 0)

    def drain(buf):
        for g in range(NG):
            pltpu.make_async_copy(table_hbm.at[idx_v.at[buf, pl.ds(g * GS, GS)]],
                                  rows_v.at[buf, pl.ds(g * GS, GS)], sems[buf]).wait()

    def compute_and_store(c, buf):
        def bag_body(b, _):
            row0 = b * F
            wv0 = w_v[buf, pl.ds(row0, LANES)]
            wv1 = w_v[buf, pl.ds(row0 + LANES, LANES)]
            acc = [jnp.zeros((LANES,), jnp.float32) for _ in range(4)]
            for f in range(F):
                ws = wv0[f] if f < LANES else wv1[f - LANES]
                for d in range(4):
                    acc[d] = acc[d] + ws * rows_v[buf, row0 + f, pl.ds(d * LANES, LANES)]
            for d in range(4):
                out_v[b, pl.ds(d * LANES, LANES)] = acc[d]
            return 0

        lax.fori_loop(0, NB, bag_body, 0)
        pltpu.sync_copy(out_v, out_hbm.at[pl.ds(w_base + c * NB, NB)])

    load_and_fire(0, 0)

    def outer(g, _):
        for j in range(2):
            c = g * 2 + j

            @pl.when(c + 1 < NCHUNK)
            def _():
                load_and_fire(c + 1, 1 - j)

            drain(j)
            compute_and_store(c, j)
        return 0

    lax.fori_loop(0, NCHUNK // 2, outer, 0)


@jax.jit
def _embed_bag(idx1, val1, msk1, table):
    mesh = plsc.VectorSubcoreMesh(core_axis_name="c", subcore_axis_name="s")
    run = functools.partial(
        pl.kernel,
        mesh=mesh,
        compiler_params=pltpu.CompilerParams(use_tc_tiling_on_sc=False),
        out_type=jax.ShapeDtypeStruct((N, D), jnp.float32),
        scratch_types=[
            pltpu.VMEM((2, CH), jnp.int32),
            pltpu.VMEM((CH,), jnp.float32),
            pltpu.VMEM((CH,), jnp.float32),
            pltpu.VMEM((2, CH + LANES), jnp.float32),
            pltpu.VMEM((2, CH, D), jnp.float32),
            pltpu.VMEM((NB, D), jnp.float32),
            pltpu.SemaphoreType.DMA,
            pltpu.SemaphoreType.DMA,
        ],
    )(_body)
    return run(idx1, val1, msk1, table)


def kernel(dynamic_indices, dynamic_values, dynamic_values_mask, embed_table):
    idx1 = dynamic_indices.reshape(-1).astype(jnp.int32)
    val1 = dynamic_values.reshape(-1)
    msk1 = dynamic_values_mask.reshape(-1).astype(jnp.float32)
    out = _embed_bag(idx1, val1, msk1, embed_table)
    return out.reshape(B, L, D)
